# baseline (speedup 1.0000x reference)
import functools

import jax
import jax.numpy as jnp
from jax import lax
from jax.experimental import pallas as pl
from jax.experimental.pallas import tpu as pltpu
from jax.experimental.pallas import tpu_sc as plsc

N_NODES = 10000
D_FEAT = 128
N_EDGES = 320000
NC = 2
NS = 16
NW = NC * NS
EPW = N_EDGES // NW
CH = 80
NCHUNK = EPW // CH
RPT = N_NODES // NS
LANES = 16
NVG = EPW // LANES

_mesh = plsc.VectorSubcoreMesh(core_axis_name="c", subcore_axis_name="s")
_sc_params = pltpu.CompilerParams(needs_layout_passes=False,
                                  use_tc_tiling_on_sc=False)


@functools.partial(
    pl.kernel, mesh=_mesh,
    out_type=jax.ShapeDtypeStruct((NW, N_NODES), jnp.float32),
    scratch_types=[
        pltpu.VMEM((EPW,), jnp.int32),
        pltpu.VMEM((N_NODES,), jnp.float32),
    ],
    compiler_params=_sc_params,
)
def _hist_cnt(rows_hbm, out_hbm, rows_v, part_v):
    cid = lax.axis_index("c")
    sid = lax.axis_index("s")
    w = cid * NS + sid
    pltpu.sync_copy(rows_hbm.at[w], rows_v)

    def zero(i, c):
        part_v[pl.ds(i * LANES, LANES)] = jnp.zeros((LANES,), jnp.float32)
        return c
    lax.fori_loop(0, N_NODES // LANES, zero, 0)

    ones = jnp.full((LANES,), 1.0, jnp.float32)

    def body(i, c):
        rv = rows_v[pl.ds(i * LANES, LANES)]
        plsc.addupdate_scatter(part_v, [rv], ones)
        return c
    lax.fori_loop(0, NVG, body, 0)
    pltpu.sync_copy(part_v, out_hbm.at[w])


def _make_spmm(with_hist: bool, dtype=jnp.float32):
    out_type = [jax.ShapeDtypeStruct((NC, N_NODES, D_FEAT), dtype)]
    scratch = [
        pltpu.VMEM((NCHUNK, CH), jnp.int32),
        pltpu.VMEM((NCHUNK, CH), jnp.int32),
        pltpu.VMEM((CH, D_FEAT), dtype),
        pltpu.VMEM((CH, D_FEAT), dtype),
        pltpu.VMEM((NCHUNK, D_FEAT), dtype),
        pltpu.VMEM_SHARED((N_NODES, D_FEAT), dtype),
        pltpu.SemaphoreType.DMA,
        pltpu.SemaphoreType.DMA,
    ]
    if with_hist:
        out_type.append(jax.ShapeDtypeStruct((NW, N_NODES), jnp.float32))
        scratch += [
            pltpu.VMEM((CH,), jnp.float32),
            pltpu.VMEM((CH,), jnp.float32),
            pltpu.VMEM((N_NODES,), jnp.float32),
            pltpu.SemaphoreType.DMA,
            pltpu.SemaphoreType.DMA,
        ]

    @functools.partial(
        pl.kernel, mesh=_mesh,
        out_type=tuple(out_type) if with_hist else out_type[0],
        scratch_types=scratch,
        compiler_params=_sc_params,
    )
    def spmm(*args):
        if with_hist:
            (rows_hbm, cols_hbm, m_hbm, tab_hbm,
             out_hbm, hout_hbm,
             rows_v, cols_v, gbuf_a, gbuf_b, zbuf, acc, sem_a, sem_b,
             qbuf_a, qbuf_b, part_v, qsem_a, qsem_b) = args
        else:
            (rows_hbm, cols_hbm, m_hbm,
             out_hbm,
             rows_v, cols_v, gbuf_a, gbuf_b, zbuf, acc, sem_a, sem_b) = args
            qbuf_a = qbuf_b = qsem_a = qsem_b = None
        cid = lax.axis_index("c")
        sid = lax.axis_index("s")
        w = cid * NS + sid
        pltpu.sync_copy(rows_hbm.at[cid, sid], rows_v)
        pltpu.sync_copy(cols_hbm.at[cid, sid], cols_v)
        if with_hist:
            def zero(i, c):
                part_v[pl.ds(i * LANES, LANES)] = jnp.zeros((LANES,), jnp.float32)
                return c
            lax.fori_loop(0, N_NODES // LANES, zero, 0)

        zn = 32 if dtype == jnp.bfloat16 else LANES
        zvec = jnp.zeros((zn,), dtype)
        zpr = D_FEAT // zn

        def zzero(i, c):
            zbuf[i // zpr, pl.ds((i % zpr) * zn, zn)] = zvec
            return c
        lax.fori_loop(0, NCHUNK * zpr, zzero, 0)
        for r in range(RPT // NCHUNK):
            pltpu.sync_copy(zbuf, acc.at[pl.ds(sid * RPT + r * NCHUNK, NCHUNK)])
        plsc.subcore_barrier()

        last = NCHUNK - 1
        gpc = CH // LANES

        def start(c, gbuf, sem, qbuf, qsem):
            pltpu.async_copy(m_hbm.at[cols_v.at[c]], gbuf, sem)
            if with_hist:
                pltpu.async_copy(tab_hbm.at[cols_v.at[c]], qbuf, qsem)

        def step(c, nxt, gbuf, sem, qbuf, qsem):
            pltpu.make_async_copy(m_hbm.at[cols_v.at[c]], gbuf, sem).wait()
            pltpu.sync_copy(gbuf, acc.at[rows_v.at[c]], add=True)
            pltpu.async_copy(m_hbm.at[cols_v.at[nxt]], gbuf, sem)
            if with_hist:
                pltpu.make_async_copy(tab_hbm.at[cols_v.at[c]], qbuf, qsem).wait()
                hist_chunk(c, qbuf)
                pltpu.async_copy(tab_hbm.at[cols_v.at[nxt]], qbuf, qsem)

        def hist_chunk(c, qbuf):
            def hbody(j, cc):
                rv = rows_v[c, pl.ds(j * LANES, LANES)]
                vv = qbuf[pl.ds(j * LANES, LANES)]
                plsc.addupdate_scatter(part_v, [rv], vv)
                return cc
            lax.fori_loop(0, gpc, hbody, 0)

        start(0, gbuf_a, sem_a, qbuf_a, qsem_a)
        start(1, gbuf_b, sem_b, qbuf_b, qsem_b)

        def body(k, carry):
            c0 = 2 * k
            c1 = 2 * k + 1
            step(c0, jnp.minimum(c0 + 2, last), gbuf_a, sem_a, qbuf_a, qsem_a)
            step(c1, jnp.minimum(c1 + 2, last), gbuf_b, sem_b, qbuf_b, qsem_b)
            return carry
        lax.fori_loop(0, (NCHUNK - 1) // 2, body, 0)

        pltpu.make_async_copy(m_hbm.at[cols_v.at[last]], gbuf_a, sem_a).wait()
        pltpu.sync_copy(gbuf_a, acc.at[rows_v.at[last]], add=True)
        if with_hist:
            pltpu.make_async_copy(tab_hbm.at[cols_v.at[last]], qbuf_a, qsem_a).wait()
            hist_chunk(last, qbuf_a)
        pltpu.make_async_copy(m_hbm.at[cols_v.at[last]], gbuf_b, sem_b).wait()
        if with_hist:
            pltpu.make_async_copy(tab_hbm.at[cols_v.at[last]], qbuf_b, qsem_b).wait()

        if with_hist:
            pltpu.sync_copy(part_v, hout_hbm.at[w])

        plsc.subcore_barrier()
        pltpu.sync_copy(acc.at[pl.ds(sid * RPT, RPT)],
                        out_hbm.at[cid, pl.ds(sid * RPT, RPT)])

    return spmm


_spmm_plain = _make_spmm(False, jnp.bfloat16)
_spmm_hist = _make_spmm(True, jnp.bfloat16)


def _tc1_body(x_ref, w_ref, pc_ref, hh_ref, hhb_ref, dinv_ref, cnt_ref):
    cnt = jnp.sum(pc_ref[...], axis=0)
    dinv = lax.rsqrt(cnt + 1.0)
    h = jnp.dot(x_ref[...], w_ref[...],
                preferred_element_type=jnp.float32,
                precision=lax.Precision.HIGHEST)
    hh = h * dinv[:, None]
    hh_ref[...] = hh
    hhb_ref[...] = hh.astype(jnp.bfloat16)
    dinv_ref[...] = dinv
    cnt_ref[...] = cnt


def _tc1(X, W, pc):
    return pl.pallas_call(
        _tc1_body,
        out_shape=[
            jax.ShapeDtypeStruct((N_NODES, D_FEAT), jnp.float32),
            jax.ShapeDtypeStruct((N_NODES, D_FEAT), jnp.bfloat16),
            jax.ShapeDtypeStruct((N_NODES,), jnp.float32),
            jax.ShapeDtypeStruct((N_NODES,), jnp.float32),
        ],
    )(X, W, pc)


def _tc2_body(tp_ref, hh_ref, dinv_ref, b_ref, xc_ref, xcb_ref, q_ref):
    t = (tp_ref[0].astype(jnp.float32) + tp_ref[1].astype(jnp.float32))
    xc = jnp.maximum(dinv_ref[...][:, None] * (t + hh_ref[...]) + b_ref[...][None, :], 0.0)
    xc_ref[...] = xc
    xcb_ref[...] = xc.astype(jnp.bfloat16)
    q_ref[...] = jnp.sum(xc * xc, axis=1)


def _tc2(tp, hh, dinv, b):
    return pl.pallas_call(
        _tc2_body,
        out_shape=[
            jax.ShapeDtypeStruct((N_NODES, D_FEAT), jnp.float32),
            jax.ShapeDtypeStruct((N_NODES, D_FEAT), jnp.bfloat16),
            jax.ShapeDtypeStruct((N_NODES,), jnp.float32),
        ],
    )(tp, hh, dinv, b)


def _tc3_body(sp_ref, xc_ref, q_ref, cnt_ref, qp_ref, out_ref):
    s = sp_ref[0].astype(jnp.float32) + sp_ref[1].astype(jnp.float32)
    qq = jnp.sum(qp_ref[...], axis=0)
    dots = jnp.sum(xc_ref[...] * s, axis=1)
    cnt = cnt_ref[...]
    sums = cnt * q_ref[...] - 2.0 * dots + qq
    out_ref[...] = jnp.tanh(sums / jnp.maximum(cnt, 1.0))


def _tc3(sp, xc, q, cnt, qp):
    return pl.pallas_call(
        _tc3_body,
        out_shape=jax.ShapeDtypeStruct((N_NODES,), jnp.float32),
    )(sp, xc, q, cnt, qp)


def kernel(X, edge_index, W, b):
    rows2 = edge_index[0].reshape(NW, EPW)
    rows4 = edge_index[0].reshape(NC, NS, NCHUNK, CH)
    cols4 = edge_index[1].reshape(NC, NS, NCHUNK, CH)

    pc = _hist_cnt(rows2)
    hh, hhb, dinv, cnt = _tc1(X, W, pc)
    tp = _spmm_plain(rows4, cols4, hhb)
    xc, xcb, q = _tc2(tp, hh, dinv, b)
    sp, qp = _spmm_hist(rows4, cols4, xcb, q)
    return _tc3(sp, xc, q, cnt, qp)


# speedup vs baseline: 1.0103x; 1.0103x over previous
import functools

import jax
import jax.numpy as jnp
from jax import lax
from jax.experimental import pallas as pl
from jax.experimental.pallas import tpu as pltpu
from jax.experimental.pallas import tpu_sc as plsc

N_NODES = 10000
D_FEAT = 128
N_EDGES = 320000
NC = 2
NS = 16
NW = NC * NS
EPW = N_EDGES // NW
CH = 80
NCHUNK = EPW // CH
RPT = N_NODES // NS
LANES = 16
NVG = EPW // LANES

_mesh = plsc.VectorSubcoreMesh(core_axis_name="c", subcore_axis_name="s")
_sc_params = pltpu.CompilerParams(needs_layout_passes=False,
                                  use_tc_tiling_on_sc=False)


@functools.partial(
    pl.kernel, mesh=_mesh,
    out_type=jax.ShapeDtypeStruct((NW, N_NODES), jnp.float32),
    scratch_types=[
        pltpu.VMEM((EPW,), jnp.int32),
        pltpu.VMEM((N_NODES,), jnp.float32),
    ],
    compiler_params=_sc_params,
)
def _hist_cnt(rows_hbm, out_hbm, rows_v, part_v):
    cid = lax.axis_index("c")
    sid = lax.axis_index("s")
    w = cid * NS + sid
    pltpu.sync_copy(rows_hbm.at[w], rows_v)

    def zero(i, c):
        part_v[pl.ds(i * LANES, LANES)] = jnp.zeros((LANES,), jnp.float32)
        return c
    lax.fori_loop(0, N_NODES // LANES, zero, 0)

    ones = jnp.full((LANES,), 1.0, jnp.float32)

    def body(i, c):
        rv = rows_v[pl.ds(i * LANES, LANES)]
        plsc.addupdate_scatter(part_v, [rv], ones)
        return c
    lax.fori_loop(0, NVG, body, 0)
    pltpu.sync_copy(part_v, out_hbm.at[w])


def _make_spmm(with_hist: bool, dtype=jnp.float32):
    out_type = [jax.ShapeDtypeStruct((NC, N_NODES, D_FEAT), dtype)]
    scratch = [
        pltpu.VMEM((NCHUNK, CH), jnp.int32),
        pltpu.VMEM((NCHUNK, CH), jnp.int32),
        pltpu.VMEM((CH, D_FEAT), dtype),
        pltpu.VMEM((CH, D_FEAT), dtype),
        pltpu.VMEM((NCHUNK, D_FEAT), dtype),
        pltpu.VMEM_SHARED((N_NODES, D_FEAT), dtype),
        pltpu.SemaphoreType.DMA,
        pltpu.SemaphoreType.DMA,
    ]
    if with_hist:
        out_type.append(jax.ShapeDtypeStruct((NW, N_NODES), jnp.float32))
        scratch += [
            pltpu.VMEM((N_NODES,), jnp.float32),
            pltpu.VMEM((N_NODES,), jnp.float32),
        ]

    @functools.partial(
        pl.kernel, mesh=_mesh,
        out_type=tuple(out_type) if with_hist else out_type[0],
        scratch_types=scratch,
        compiler_params=_sc_params,
    )
    def spmm(*args):
        if with_hist:
            (rows_hbm, cols_hbm, m_hbm, tab_hbm,
             out_hbm, hout_hbm,
             rows_v, cols_v, gbuf_a, gbuf_b, zbuf, acc, sem_a, sem_b,
             tab_v, part_v) = args
        else:
            (rows_hbm, cols_hbm, m_hbm,
             out_hbm,
             rows_v, cols_v, gbuf_a, gbuf_b, zbuf, acc, sem_a, sem_b) = args
        cid = lax.axis_index("c")
        sid = lax.axis_index("s")
        w = cid * NS + sid
        pltpu.sync_copy(rows_hbm.at[cid, sid], rows_v)
        pltpu.sync_copy(cols_hbm.at[cid, sid], cols_v)
        if with_hist:
            pltpu.sync_copy(tab_hbm, tab_v)

            def zero(i, c):
                part_v[pl.ds(i * LANES, LANES)] = jnp.zeros((LANES,), jnp.float32)
                return c
            lax.fori_loop(0, N_NODES // LANES, zero, 0)

        zn = 32 if dtype == jnp.bfloat16 else LANES
        zvec = jnp.zeros((zn,), dtype)
        zpr = D_FEAT // zn

        def zzero(i, c):
            zbuf[i // zpr, pl.ds((i % zpr) * zn, zn)] = zvec
            return c
        lax.fori_loop(0, NCHUNK * zpr, zzero, 0)
        for r in range(RPT // NCHUNK):
            pltpu.sync_copy(zbuf, acc.at[pl.ds(sid * RPT + r * NCHUNK, NCHUNK)])
        plsc.subcore_barrier()

        last = NCHUNK - 1
        gpc = CH // LANES

        def start(c, gbuf, sem):
            pltpu.async_copy(m_hbm.at[cols_v.at[c]], gbuf, sem)

        def step(c, nxt, gbuf, sem):
            pltpu.make_async_copy(m_hbm.at[cols_v.at[c]], gbuf, sem).wait()
            pltpu.sync_copy(gbuf, acc.at[rows_v.at[c]], add=True)
            pltpu.async_copy(m_hbm.at[cols_v.at[nxt]], gbuf, sem)
            if with_hist:
                hist_chunk(c)

        def hist_chunk(c):
            def hbody(j, cc):
                rv = rows_v[c, pl.ds(j * LANES, LANES)]
                cv = cols_v[c, pl.ds(j * LANES, LANES)]
                vv = plsc.load_gather(tab_v, [cv])
                plsc.addupdate_scatter(part_v, [rv], vv)
                return cc
            lax.fori_loop(0, gpc, hbody, 0)

        start(0, gbuf_a, sem_a)
        start(1, gbuf_b, sem_b)

        def body(k, carry):
            c0 = 2 * k
            c1 = 2 * k + 1
            step(c0, jnp.minimum(c0 + 2, last), gbuf_a, sem_a)
            step(c1, jnp.minimum(c1 + 2, last), gbuf_b, sem_b)
            return carry
        lax.fori_loop(0, (NCHUNK - 1) // 2, body, 0)

        pltpu.make_async_copy(m_hbm.at[cols_v.at[last]], gbuf_a, sem_a).wait()
        pltpu.sync_copy(gbuf_a, acc.at[rows_v.at[last]], add=True)
        if with_hist:
            hist_chunk(last)
        pltpu.make_async_copy(m_hbm.at[cols_v.at[last]], gbuf_b, sem_b).wait()

        if with_hist:
            pltpu.sync_copy(part_v, hout_hbm.at[w])

        plsc.subcore_barrier()
        pltpu.sync_copy(acc.at[pl.ds(sid * RPT, RPT)],
                        out_hbm.at[cid, pl.ds(sid * RPT, RPT)])

    return spmm


_spmm_plain = _make_spmm(False, jnp.bfloat16)
_spmm_hist = _make_spmm(True, jnp.bfloat16)


def _tc0_body(x_ref, w_ref, h_ref):
    h_ref[...] = jnp.dot(x_ref[...], w_ref[...],
                         preferred_element_type=jnp.float32,
                         precision=lax.Precision.HIGHEST)


def _tc0(X, W):
    return pl.pallas_call(
        _tc0_body,
        out_shape=jax.ShapeDtypeStruct((N_NODES, D_FEAT), jnp.float32),
    )(X, W)


def _tc1_body(h_ref, pc_ref, hh_ref, hhb_ref, dinv_ref, cnt_ref):
    cnt = jnp.sum(pc_ref[...], axis=0)
    dinv = lax.rsqrt(cnt + 1.0)
    hh = h_ref[...] * dinv[:, None]
    hh_ref[...] = hh
    hhb_ref[...] = hh.astype(jnp.bfloat16)
    dinv_ref[...] = dinv
    cnt_ref[...] = cnt


def _tc1(h, pc):
    return pl.pallas_call(
        _tc1_body,
        out_shape=[
            jax.ShapeDtypeStruct((N_NODES, D_FEAT), jnp.float32),
            jax.ShapeDtypeStruct((N_NODES, D_FEAT), jnp.bfloat16),
            jax.ShapeDtypeStruct((N_NODES,), jnp.float32),
            jax.ShapeDtypeStruct((N_NODES,), jnp.float32),
        ],
    )(h, pc)


def _tc2_body(tp_ref, hh_ref, dinv_ref, b_ref, xc_ref, xcb_ref, q_ref):
    t = (tp_ref[0].astype(jnp.float32) + tp_ref[1].astype(jnp.float32))
    xc = jnp.maximum(dinv_ref[...][:, None] * (t + hh_ref[...]) + b_ref[...][None, :], 0.0)
    xc_ref[...] = xc
    xcb_ref[...] = xc.astype(jnp.bfloat16)
    q_ref[...] = jnp.sum(xc * xc, axis=1)


def _tc2(tp, hh, dinv, b):
    return pl.pallas_call(
        _tc2_body,
        out_shape=[
            jax.ShapeDtypeStruct((N_NODES, D_FEAT), jnp.float32),
            jax.ShapeDtypeStruct((N_NODES, D_FEAT), jnp.bfloat16),
            jax.ShapeDtypeStruct((N_NODES,), jnp.float32),
        ],
    )(tp, hh, dinv, b)


def _tc3_body(sp_ref, xc_ref, q_ref, cnt_ref, qp_ref, out_ref):
    s = sp_ref[0].astype(jnp.float32) + sp_ref[1].astype(jnp.float32)
    qq = jnp.sum(qp_ref[...], axis=0)
    dots = jnp.sum(xc_ref[...] * s, axis=1)
    cnt = cnt_ref[...]
    sums = cnt * q_ref[...] - 2.0 * dots + qq
    out_ref[...] = jnp.tanh(sums / jnp.maximum(cnt, 1.0))


def _tc3(sp, xc, q, cnt, qp):
    return pl.pallas_call(
        _tc3_body,
        out_shape=jax.ShapeDtypeStruct((N_NODES,), jnp.float32),
    )(sp, xc, q, cnt, qp)


def kernel(X, edge_index, W, b):
    rows2 = edge_index[0].reshape(NW, EPW)
    rows4 = edge_index[0].reshape(NC, NS, NCHUNK, CH)
    cols4 = edge_index[1].reshape(NC, NS, NCHUNK, CH)

    pc = _hist_cnt(rows2)
    h = _tc0(X, W)
    hh, hhb, dinv, cnt = _tc1(h, pc)
    tp = _spmm_plain(rows4, cols4, hhb)
    xc, xcb, q = _tc2(tp, hh, dinv, b)
    sp, qp = _spmm_hist(rows4, cols4, xcb, q)
    return _tc3(sp, xc, q, cnt, qp)
